# Initial kernel scaffold; baseline (speedup 1.0000x reference)
import jax
import jax.numpy as jnp
from jax.experimental import pallas as pl


def kernel(point_cloud_xyz, params):
    raise NotImplementedError("write your pallas kernel here")


# speedup vs baseline: 3.2659x; 3.2659x over previous
import functools

import jax
import jax.numpy as jnp
import numpy as np
from jax import lax
from jax.experimental import pallas as pl
from jax.experimental.pallas import tpu as pltpu


def _fps_body(xt_ref, out_ref, *, npoint, n):
    xtx = xt_ref[:, 0, :]
    xty = xt_ref[:, 1, :]
    xtz = xt_ref[:, 2, :]
    B = xtx.shape[0]
    iota = lax.broadcasted_iota(jnp.int32, (B, n), 1)

    def body(i, state):
        distance, farthest = state
        sel = (iota == farthest).astype(jnp.float32)
        cx = jnp.sum(sel * xtx, axis=1, keepdims=True)
        cy = jnp.sum(sel * xty, axis=1, keepdims=True)
        cz = jnp.sum(sel * xtz, axis=1, keepdims=True)
        out_ref[:, pl.ds(i, 1), :] = jnp.concatenate(
            [cx, cy, cz], axis=1)[:, None, :]
        dx = xtx - cx
        dy = xty - cy
        dz = xtz - cz
        dist = (dx * dx + dy * dy) + dz * dz
        distance = jnp.minimum(distance, dist)
        m = jnp.max(distance, axis=1, keepdims=True)
        idx = jnp.min(jnp.where(distance == m, iota, jnp.int32(n)),
                      axis=1, keepdims=True)
        return distance, idx

    init = (jnp.full((B, n), 1e10, jnp.float32),
            jnp.zeros((B, 1), jnp.int32))
    lax.fori_loop(0, npoint, body, init)


def _fps(xyzT, npoint):
    B, _, n = xyzT.shape
    return pl.pallas_call(
        functools.partial(_fps_body, npoint=npoint, n=n),
        out_shape=jax.ShapeDtypeStruct((B, npoint, 3), jnp.float32),
    )(xyzT)


def _cumsum_lanes(x, n):
    s = 1
    while s < n:
        shifted = jnp.concatenate(
            [jnp.zeros_like(x[:, :s]), x[:, :-s]], axis=1)
        x = x + shifted
        s *= 2
    return x


def _group_body(xt_ref, pt_ref, new_ref, gf_ref, gx_ref, *, r2, K, n, SB, C):
    xt = xt_ref[0]
    nw = new_ref[0]
    dx = nw[:, 0:1] - xt[0:1, :]
    dy = nw[:, 1:2] - xt[1:2, :]
    dz = nw[:, 2:3] - xt[2:3, :]
    d = (dx * dx + dy * dy) + dz * dz
    mask = d <= r2
    c = _cumsum_lanes(mask.astype(jnp.float32), n)
    ci = c.astype(jnp.int32)
    count = ci[:, n - 1:n]
    first = mask & (ci == 1)
    kv = lax.broadcasted_iota(jnp.int32, (SB, K, 1), 1)
    selk = mask[:, None, :] & ((ci - 1)[:, None, :] == kv)
    padk = kv >= count[:, :, None]
    Mb = (selk | (padk & first[:, None, :])).astype(jnp.float32)
    Mb = Mb.reshape(SB * K, n)
    dn = (((1,), (1,)), ((), ()))
    gf = lax.dot_general(Mb, pt_ref[0], dn,
                         precision=lax.Precision.HIGHEST,
                         preferred_element_type=jnp.float32)
    gx = lax.dot_general(Mb, xt, dn, precision=lax.Precision.HIGHEST,
                         preferred_element_type=jnp.float32)
    gf_ref[0] = gf.reshape(SB, K, C)
    gx_ref[0] = gx.reshape(SB, K, 3) - nw[:, None, :]


def _group(xyzT, ptsT, new_xyz, radius, K, SB=8):
    B, _, n = xyzT.shape
    C = ptsT.shape[1]
    S = new_xyz.shape[1]
    r2 = np.float32(radius ** 2)
    return pl.pallas_call(
        functools.partial(_group_body, r2=r2, K=K, n=n, SB=SB, C=C),
        grid=(B, S // SB),
        in_specs=[
            pl.BlockSpec((1, 3, n), lambda b, s: (b, 0, 0)),
            pl.BlockSpec((1, C, n), lambda b, s: (b, 0, 0)),
            pl.BlockSpec((1, SB, 3), lambda b, s: (b, s, 0)),
        ],
        out_specs=[
            pl.BlockSpec((1, SB, K, C), lambda b, s: (b, s, 0, 0)),
            pl.BlockSpec((1, SB, K, 3), lambda b, s: (b, s, 0, 0)),
        ],
        out_shape=[
            jax.ShapeDtypeStruct((B, S, K, C), jnp.float32),
            jax.ShapeDtypeStruct((B, S, K, 3), jnp.float32),
        ],
    )(xyzT, ptsT, new_xyz)


def _mm_first_body(x_ref, W_ref, b_ref, y_ref):
    y_ref[...] = jnp.dot(x_ref[...], W_ref[...],
                         preferred_element_type=jnp.float32) + b_ref[...]


def _mm_first(x, W, b):
    R, Cin = x.shape
    Cout = W.shape[1]
    RB = min(1024, R)
    return pl.pallas_call(
        _mm_first_body,
        grid=(R // RB,),
        in_specs=[
            pl.BlockSpec((RB, Cin), lambda i: (i, 0)),
            pl.BlockSpec((Cin, Cout), lambda i: (0, 0)),
            pl.BlockSpec((1, Cout), lambda i: (0, 0)),
        ],
        out_specs=pl.BlockSpec((RB, Cout), lambda i: (i, 0)),
        out_shape=jax.ShapeDtypeStruct((R, Cout), jnp.float32),
    )(x, W, b)


def _normed(x, st, g_ref, be_ref):
    mean = st[0:1, :]
    var = st[1:2, :]
    t = (x - mean) / jnp.sqrt(var + 1e-5) * g_ref[...] + be_ref[...]
    return jnp.maximum(t, 0.0)


def _maxpool_body(t_ref, o_ref, *, K, RB, C):
    o_ref[...] = jnp.max(t_ref[...].reshape(RB // K, K, C), axis=1)


def _maxpool(t, K):
    R, C = t.shape
    RB = min(1024, R)
    return pl.pallas_call(
        functools.partial(_maxpool_body, K=K, RB=RB, C=C),
        grid=(R // RB,),
        in_specs=[pl.BlockSpec((RB, C), lambda i: (i, 0))],
        out_specs=pl.BlockSpec((RB // K, C), lambda i: (i, 0)),
        out_shape=jax.ShapeDtypeStruct((R // K, C), jnp.float32),
    )(t)


def _interp_body(x1_ref, x2_ref, p2_ref, o_ref, *, s, NB):
    x1 = x1_ref[0]
    x2 = x2_ref[0]
    dx = x1[:, 0:1] - x2[0:1, :]
    dy = x1[:, 1:2] - x2[1:2, :]
    dz = x1[:, 2:3] - x2[2:3, :]
    d = (dx * dx + dy * dy) + dz * dz
    iota = lax.broadcasted_iota(jnp.int32, (NB, s), 1)
    dcur = d
    mins, ohs = [], []
    for k in range(3):
        m = jnp.min(dcur, axis=1, keepdims=True)
        idx = jnp.min(jnp.where(dcur == m, iota, jnp.int32(s)),
                      axis=1, keepdims=True)
        oh = iota == idx
        mins.append(m)
        ohs.append(oh)
        if k < 2:
            dcur = jnp.where(oh, jnp.float32(1e30), dcur)
    r0 = 1.0 / (mins[0] + 1e-8)
    r1 = 1.0 / (mins[1] + 1e-8)
    r2 = 1.0 / (mins[2] + 1e-8)
    rs = (r0 + r1) + r2
    Wm = ((r0 / rs) * ohs[0].astype(jnp.float32)
          + (r1 / rs) * ohs[1].astype(jnp.float32)
          + (r2 / rs) * ohs[2].astype(jnp.float32))
    o_ref[0] = jnp.dot(Wm, p2_ref[0], precision=lax.Precision.HIGHEST,
                       preferred_element_type=jnp.float32)


def _interp(xyz1, xyz2T, points2):
    B, n, _ = xyz1.shape
    s = xyz2T.shape[2]
    C2 = points2.shape[2]
    NB = min(256, n)
    return pl.pallas_call(
        functools.partial(_interp_body, s=s, NB=NB),
        grid=(B, n // NB),
        in_specs=[
            pl.BlockSpec((1, NB, 3), lambda b, i: (b, i, 0)),
            pl.BlockSpec((1, 3, s), lambda b, i: (b, 0, 0)),
            pl.BlockSpec((1, s, C2), lambda b, i: (b, 0, 0)),
        ],
        out_specs=pl.BlockSpec((1, NB, C2), lambda b, i: (b, i, 0)),
        out_shape=jax.ShapeDtypeStruct((B, n, C2), jnp.float32),
    )(xyz1, xyz2T, points2)


def _head_body(t_ref, W_ref, b_ref, o_ref):
    y = jnp.dot(t_ref[...], W_ref[...],
                preferred_element_type=jnp.float32) + b_ref[...]
    o_ref[...] = jax.nn.sigmoid(y)


def _head_final(t, W2, b2):
    R, C = t.shape
    RB = min(1024, R)
    return pl.pallas_call(
        _head_body,
        grid=(R // RB,),
        in_specs=[
            pl.BlockSpec((RB, C), lambda i: (i, 0)),
            pl.BlockSpec((C, 1), lambda i: (0, 0)),
            pl.BlockSpec((1, 1), lambda i: (0, 0)),
        ],
        out_specs=pl.BlockSpec((RB, 1), lambda i: (i, 0)),
        out_shape=jax.ShapeDtypeStruct((R, 1), jnp.float32),
    )(t, W2, b2)


def _row2(v):
    return v.reshape(1, -1)


def _xla_norm(y, g, be):
    mean = jnp.mean(y, axis=0, keepdims=True)
    var = jnp.var(y, axis=0, keepdims=True)
    return jax.nn.relu((y - mean) / jnp.sqrt(var + 1e-5) * g + be)


def _sa_msg(xyz, points, npoint, radius_list, nsample_list, branch_params):
    B, n, _ = xyz.shape
    C = points.shape[-1]
    xyzT = jnp.transpose(xyz, (0, 2, 1))
    ptsT = jnp.transpose(points, (0, 2, 1))
    new_xyz = _fps(xyzT, npoint)
    outs = []
    for radius, K, mlps in zip(radius_list, nsample_list, branch_params):
        gf, gx = _group(xyzT, ptsT, new_xyz, radius, K)
        R = B * npoint * K
        x = jnp.concatenate([gf, gx], axis=-1).reshape(R, C + 3)
        W0, b0, g0, e0 = mlps[0]
        y = _mm_first(x, W0, _row2(b0))
        t = _xla_norm(y, g0, e0)
        for (W, bb, g, be) in mlps[1:]:
            y = _mm_first(t, W, _row2(bb))
            t = _xla_norm(y, g, be)
        out = _maxpool(t, K)
        outs.append(out.reshape(B, npoint, -1))
    return new_xyz, jnp.concatenate(outs, axis=-1)


def _fp(xyz1, xyz2, points1, points2, mlps):
    B, n, _ = xyz1.shape
    interp = _interp(xyz1, jnp.transpose(xyz2, (0, 2, 1)), points2)
    R = B * n
    C2 = interp.shape[-1]
    W0, b0, g0, e0 = mlps[0]
    if points1 is None:
        x = interp.reshape(R, C2)
    else:
        x = jnp.concatenate([points1, interp], axis=-1).reshape(
            R, points1.shape[-1] + C2)
    y = _mm_first(x, W0, _row2(b0))
    t = _xla_norm(y, g0, e0)
    for (W, bb, g, be) in mlps[1:]:
        y = _mm_first(t, W, _row2(bb))
        t = _xla_norm(y, g, be)
    out = t
    return out.reshape(B, n, -1)


def kernel(point_cloud_xyz, params):
    B = point_cloud_xyz.shape[0]
    N = point_cloud_xyz.shape[2]
    l0_xyz = jnp.transpose(point_cloud_xyz[:, :3, :], (0, 2, 1))
    l0_points = jnp.transpose(point_cloud_xyz, (0, 2, 1))

    l1_xyz, l1_points = _sa_msg(l0_xyz, l0_points, 1024, (0.05, 0.1),
                                (16, 32), params['sa1'])
    l2_xyz, l2_points = _sa_msg(l1_xyz, l1_points, 256, (0.1, 0.2),
                                (16, 32), params['sa2'])
    l3_xyz, l3_points = _sa_msg(l2_xyz, l2_points, 64, (0.2, 0.4),
                                (16, 32), params['sa3'])
    l4_xyz, l4_points = _sa_msg(l3_xyz, l3_points, 16, (0.4, 0.8),
                                (16, 32), params['sa4'])

    l3p = _fp(l3_xyz, l4_xyz, l3_points, l4_points, params['fp4'])
    l2p = _fp(l2_xyz, l3_xyz, l2_points, l3p, params['fp3'])
    l1p = _fp(l1_xyz, l2_xyz, l1_points, l2p, params['fp2'])
    l0p = _fp(l0_xyz, l1_xyz, None, l1p, params['fp1'])

    W1, b1 = params['conv1']
    g1, be1 = params['bn1']
    R = B * N
    y = _mm_first(l0p.reshape(R, -1), W1, _row2(b1))
    t = _xla_norm(y, g1, be1)
    W2, b2 = params['conv2']
    seg = _head_final(t, W2, _row2(b2))
    return seg.reshape(B, N, 1), jnp.transpose(l4_points, (0, 2, 1))
